# baseline (speedup 1.0000x reference)
import functools

import jax
import jax.numpy as jnp
from jax import lax
from jax.experimental import pallas as pl
from jax.experimental.pallas import tpu as pltpu
from jax.experimental.pallas import tpu_sc as plsc

D = 768
N_EXP = 8
TOP_K = 2
N_TOK = 2048
CAP = 640
N_SLOT = N_EXP * CAP
SENTINEL = N_SLOT

TB = 256
STEPS = N_TOK // TB

NC, NS, L = 2, 16, 16
NW = NC * NS
ROWS_PER_W = N_SLOT // NW
GCHUNK = ROWS_PER_W // 2
ZROW = N_TOK
NPAD = 256
MAP_PAD = N_SLOT + 512
A_PER_S = N_TOK * TOP_K // NS


def _tc_body(x_ref, w_ref, gates_ref, slots_ref, mask_ref, xaug_ref, off_ref):
    step = pl.program_id(0)

    @pl.when(step == 0)
    def _():
        off_ref[...] = jnp.zeros((1, N_EXP), jnp.int32)

    x = x_ref[...]
    w = w_ref[...]

    xaug_ref[...] = jnp.where(step < STEPS, x, 0.0)

    @pl.when(step < STEPS)
    def _router_and_mask():
        _tc_router_block(x, w, gates_ref, slots_ref, mask_ref, off_ref)


def _tc_router_block(x, w, gates_ref, slots_ref, mask_ref, off_ref):
    logits = lax.dot_general(
        x.astype(jnp.bfloat16), w.astype(jnp.bfloat16),
        (((1,), (1,)), ((), ())),
        preferred_element_type=jnp.float32,
    )

    ioe = lax.broadcasted_iota(jnp.int32, (TB, N_EXP), 1)
    m0 = jnp.max(logits, axis=1, keepdims=True)
    idx0 = jnp.min(jnp.where(logits == m0, ioe, N_EXP), axis=1, keepdims=True)
    l2 = jnp.where(ioe == idx0, -jnp.inf, logits)
    m1 = jnp.max(l2, axis=1, keepdims=True)
    idx1 = jnp.min(jnp.where(l2 == m1, ioe, N_EXP), axis=1, keepdims=True)

    e1 = jnp.exp(m1 - m0)
    denom = 1.0 + e1
    g0 = 1.0 / denom
    g1 = e1 / denom

    onehot0 = ioe == idx0
    onehot1 = ioe == idx1
    cnt = onehot0.astype(jnp.int32) + onehot1.astype(jnp.int32)

    c = cnt
    d = 1
    while d < TB:
        c = c + jnp.concatenate(
            [jnp.zeros((d, N_EXP), jnp.int32), c[: TB - d, :]], axis=0)
        d *= 2
    off = off_ref[...]
    excl = c - cnt + off

    pos0 = jnp.sum(jnp.where(onehot0, excl, 0), axis=1, keepdims=True)
    pos1 = jnp.sum(jnp.where(onehot1, excl, 0), axis=1, keepdims=True)
    slot0 = jnp.where(pos0 < CAP, idx0 * CAP + pos0, SENTINEL)
    slot1 = jnp.where(pos1 < CAP, idx1 * CAP + pos1, SENTINEL)

    if gates_ref is not None:
        gates_ref[...] = jnp.concatenate([g0, g1], axis=1)
    slots_ref[...] = jnp.concatenate([slot0, slot1], axis=1)

    if mask_ref is not None:
        col = (lax.broadcasted_iota(jnp.int32, (TB, N_EXP, CAP), 1) * CAP
               + lax.broadcasted_iota(jnp.int32, (TB, N_EXP, CAP), 2))
        s0 = slot0.reshape(TB, 1, 1)
        s1 = slot1.reshape(TB, 1, 1)
        mask_ref[...] = (
            jnp.where(col == s0, g0.reshape(TB, 1, 1), 0.0)
            + jnp.where(col == s1, g1.reshape(TB, 1, 1), 0.0))

    off_ref[...] = off + jnp.sum(cnt, axis=0, keepdims=True)


def _tc_router(x_flat, w_gate):
    clamp = lambda t: jnp.minimum(t, STEPS - 1)
    return pl.pallas_call(
        _tc_body,
        grid=(STEPS + 1,),
        in_specs=[
            pl.BlockSpec((TB, D), lambda t: (jnp.minimum(t, STEPS - 1), 0)),
            pl.BlockSpec((N_EXP, D), lambda t: (0, 0)),
        ],
        out_specs=[
            pl.BlockSpec((TB, TOP_K), lambda t: (clamp(t), 0)),
            pl.BlockSpec((TB, TOP_K), lambda t: (clamp(t), 0)),
            pl.BlockSpec((TB, N_EXP, CAP), lambda t: (clamp(t), 0, 0)),
            pl.BlockSpec((TB, D), lambda t: (t, 0)),
        ],
        out_shape=[
            jax.ShapeDtypeStruct((N_TOK, TOP_K), jnp.float32),
            jax.ShapeDtypeStruct((N_TOK, TOP_K), jnp.int32),
            jax.ShapeDtypeStruct((N_TOK, N_EXP, CAP), jnp.float32),
            jax.ShapeDtypeStruct((N_TOK + NPAD, D), jnp.float32),
        ],
        scratch_shapes=[pltpu.VMEM((1, N_EXP), jnp.int32)],
    )(x_flat, w_gate)


def _sc_gather_body(x_hbm, slots_hbm, out_hbm,
                    shared_map, zbuf, tokbufs, idxbufs, gidx, gbufs,
                    gsem, wsem):
    cid = lax.axis_index("c")
    sid = lax.axis_index("s")
    wid = sid * NC + cid
    lane = lax.broadcasted_iota(jnp.int32, (L,), 0)

    zchunk = MAP_PAD // NS
    for m in range(zchunk // L):
        base = sid * zchunk + m * L
        zbuf[pl.ds(m * L, L)] = ZROW + ((base + lane) & (NPAD - 1))
    pltpu.sync_copy(zbuf, shared_map.at[pl.ds(sid * zchunk, zchunk)])

    for j in range(A_PER_S // 128):
        for m in range(128 // L):
            base = sid * A_PER_S + j * 128 + m * L
            tokbufs[j][pl.ds(m * L, L)] = lax.shift_right_logical(
                base + lane, 1)
        pltpu.sync_copy(
            slots_hbm.at[pl.ds(sid * A_PER_S + j * 128, 128)], idxbufs[j])

    plsc.subcore_barrier()
    for j in range(A_PER_S // 128):
        pltpu.sync_copy(tokbufs[j], shared_map.at[idxbufs[j]])
    plsc.subcore_barrier()

    base = wid * ROWS_PER_W
    for h in range(2):
        pltpu.sync_copy(
            shared_map.at[pl.ds(base + h * GCHUNK, GCHUNK)], gidx[h])
    g0 = pltpu.async_copy(x_hbm.at[gidx[0]], gbufs[0], gsem[0])
    g1 = pltpu.async_copy(x_hbm.at[gidx[1]], gbufs[1], gsem[1])
    g0.wait()
    w0 = pltpu.async_copy(gbufs[0], out_hbm.at[pl.ds(base, GCHUNK)], wsem[0])
    g1.wait()
    w1 = pltpu.async_copy(
        gbufs[1], out_hbm.at[pl.ds(base + GCHUNK, GCHUNK)], wsem[1])
    w0.wait()
    w1.wait()


@functools.cache
def _sc_dispatch():
    return pl.kernel(
        _sc_gather_body,
        out_type=jax.ShapeDtypeStruct((N_SLOT, D), jnp.float32),
        mesh=plsc.VectorSubcoreMesh(core_axis_name="c", subcore_axis_name="s"),
        compiler_params=pltpu.CompilerParams(needs_layout_passes=False),
        scratch_types=[
            pltpu.VMEM_SHARED((MAP_PAD,), jnp.int32),
            pltpu.VMEM((MAP_PAD // NS,), jnp.int32),
            [pltpu.VMEM((128,), jnp.int32) for _ in range(A_PER_S // 128)],
            [pltpu.VMEM((128,), jnp.int32) for _ in range(A_PER_S // 128)],
            [pltpu.VMEM((GCHUNK,), jnp.int32) for _ in range(2)],
            [pltpu.VMEM((GCHUNK, D), jnp.float32) for _ in range(2)],
            [pltpu.SemaphoreType.DMA for _ in range(2)],
            [pltpu.SemaphoreType.DMA for _ in range(2)],
        ],
    )


def _slots_body(x_ref, w_ref, slots_ref, xaug_ref, off_ref):
    step = pl.program_id(0)

    @pl.when(step == 0)
    def _():
        off_ref[...] = jnp.zeros((1, N_EXP), jnp.int32)

    x = x_ref[...]
    xaug_ref[...] = jnp.where(step < STEPS, x, 0.0)

    @pl.when(step < STEPS)
    def _slots():
        _tc_router_block(x, w_ref[...], None, slots_ref, None, off_ref)


def _tc_slots(x_flat, w_gate):
    clamp = lambda t: jnp.minimum(t, STEPS - 1)
    return pl.pallas_call(
        _slots_body,
        grid=(STEPS + 1,),
        in_specs=[
            pl.BlockSpec((TB, D), lambda t: (jnp.minimum(t, STEPS - 1), 0)),
            pl.BlockSpec((N_EXP, D), lambda t: (0, 0)),
        ],
        out_specs=[
            pl.BlockSpec((TB, TOP_K), lambda t: (clamp(t), 0)),
            pl.BlockSpec((TB, D), lambda t: (t, 0)),
        ],
        out_shape=[
            jax.ShapeDtypeStruct((N_TOK, TOP_K), jnp.int32),
            jax.ShapeDtypeStruct((N_TOK + NPAD, D), jnp.float32),
        ],
        scratch_shapes=[pltpu.VMEM((1, N_EXP), jnp.int32)],
    )(x_flat, w_gate)


def kernel(x, W_gate):
    x_flat = x.reshape(N_TOK, D)
    slots, x_aug = _tc_slots(x_flat, W_gate)
    gates, _, exp_mask, _ = _tc_router(x_flat, W_gate)
    exp_batches = _sc_dispatch()(x_aug, slots.reshape(N_TOK * TOP_K))
    return (gates, exp_mask, exp_batches.reshape(N_EXP, CAP, D))


# speedup vs baseline: 1.1550x; 1.1550x over previous
import functools

import jax
import jax.numpy as jnp
from jax import lax
from jax.experimental import pallas as pl
from jax.experimental.pallas import tpu as pltpu
from jax.experimental.pallas import tpu_sc as plsc

D = 768
N_EXP = 8
TOP_K = 2
N_TOK = 2048
CAP = 640
N_SLOT = N_EXP * CAP
SENTINEL = N_SLOT

TB = 256
STEPS = N_TOK // TB

NC, NS, L = 2, 16, 16
NW = NC * NS
ROWS_PER_W = N_SLOT // NW
GCHUNK = ROWS_PER_W // 2
ZROW = N_TOK
NPAD = 256
MAP_PAD = N_SLOT + 512
A_PER_S = N_TOK * TOP_K // NS


def _tc_body(x_ref, w_ref, gates_ref, slots_ref, mask_ref, xaug_ref, off_ref):
    step = pl.program_id(0)

    @pl.when(step == 0)
    def _():
        off_ref[...] = jnp.zeros((1, N_EXP), jnp.int32)

    x = x_ref[...]
    w = w_ref[...]

    xaug_ref[...] = jnp.where(step < STEPS, x, 0.0)

    @pl.when(step < STEPS)
    def _router_and_mask():
        _tc_router_block(x, w, gates_ref, slots_ref, mask_ref, off_ref)


def _tc_router_block(x, w, gates_ref, slots_ref, mask_ref, off_ref):
    logits = lax.dot_general(
        x.astype(jnp.bfloat16), w.astype(jnp.bfloat16),
        (((1,), (1,)), ((), ())),
        preferred_element_type=jnp.float32,
    )

    ioe = lax.broadcasted_iota(jnp.int32, (TB, N_EXP), 1)
    m0 = jnp.max(logits, axis=1, keepdims=True)
    idx0 = jnp.min(jnp.where(logits == m0, ioe, N_EXP), axis=1, keepdims=True)
    l2 = jnp.where(ioe == idx0, -jnp.inf, logits)
    m1 = jnp.max(l2, axis=1, keepdims=True)
    idx1 = jnp.min(jnp.where(l2 == m1, ioe, N_EXP), axis=1, keepdims=True)

    e1 = jnp.exp(m1 - m0)
    denom = 1.0 + e1
    g0 = 1.0 / denom
    g1 = e1 / denom

    onehot0 = ioe == idx0
    onehot1 = ioe == idx1
    cnt = onehot0.astype(jnp.int32) + onehot1.astype(jnp.int32)

    c = cnt
    d = 1
    while d < TB:
        c = c + jnp.concatenate(
            [jnp.zeros((d, N_EXP), jnp.int32), c[: TB - d, :]], axis=0)
        d *= 2
    off = off_ref[...]
    excl = c - cnt + off

    pos0 = jnp.sum(jnp.where(onehot0, excl, 0), axis=1, keepdims=True)
    pos1 = jnp.sum(jnp.where(onehot1, excl, 0), axis=1, keepdims=True)
    slot0 = jnp.where(pos0 < CAP, idx0 * CAP + pos0, SENTINEL)
    slot1 = jnp.where(pos1 < CAP, idx1 * CAP + pos1, SENTINEL)

    gates_ref[...] = jnp.concatenate([g0, g1], axis=1)
    slots_ref[...] = jnp.concatenate([slot0, slot1], axis=1)

    col = (lax.broadcasted_iota(jnp.int32, (TB, N_EXP, CAP), 1) * CAP
           + lax.broadcasted_iota(jnp.int32, (TB, N_EXP, CAP), 2))
    s0 = slot0.reshape(TB, 1, 1)
    s1 = slot1.reshape(TB, 1, 1)
    mask_ref[...] = (
        jnp.where(col == s0, g0.reshape(TB, 1, 1), 0.0)
        + jnp.where(col == s1, g1.reshape(TB, 1, 1), 0.0))

    off_ref[...] = off + jnp.sum(cnt, axis=0, keepdims=True)


def _tc_router(x_flat, w_gate):
    clamp = lambda t: jnp.minimum(t, STEPS - 1)
    return pl.pallas_call(
        _tc_body,
        grid=(STEPS + 1,),
        in_specs=[
            pl.BlockSpec((TB, D), lambda t: (jnp.minimum(t, STEPS - 1), 0)),
            pl.BlockSpec((N_EXP, D), lambda t: (0, 0)),
        ],
        out_specs=[
            pl.BlockSpec((TB, TOP_K), lambda t: (clamp(t), 0)),
            pl.BlockSpec((TB, TOP_K), lambda t: (clamp(t), 0)),
            pl.BlockSpec((TB, N_EXP, CAP), lambda t: (clamp(t), 0, 0)),
            pl.BlockSpec((TB, D), lambda t: (t, 0)),
        ],
        out_shape=[
            jax.ShapeDtypeStruct((N_TOK, TOP_K), jnp.float32),
            jax.ShapeDtypeStruct((N_TOK, TOP_K), jnp.int32),
            jax.ShapeDtypeStruct((N_TOK, N_EXP, CAP), jnp.float32),
            jax.ShapeDtypeStruct((N_TOK + NPAD, D), jnp.float32),
        ],
        scratch_shapes=[pltpu.VMEM((1, N_EXP), jnp.int32)],
    )(x_flat, w_gate)


def _sc_gather_body(x_hbm, slots_hbm, out_hbm,
                    shared_map, zbuf, tokbufs, idxbufs, gidx, gbufs,
                    gsem, wsem):
    cid = lax.axis_index("c")
    sid = lax.axis_index("s")
    wid = sid * NC + cid
    lane = lax.broadcasted_iota(jnp.int32, (L,), 0)

    zchunk = MAP_PAD // NS
    for m in range(zchunk // L):
        base = sid * zchunk + m * L
        zbuf[pl.ds(m * L, L)] = ZROW + ((base + lane) & (NPAD - 1))
    pltpu.sync_copy(zbuf, shared_map.at[pl.ds(sid * zchunk, zchunk)])

    for j in range(A_PER_S // 128):
        for m in range(128 // L):
            base = sid * A_PER_S + j * 128 + m * L
            tokbufs[j][pl.ds(m * L, L)] = lax.shift_right_logical(
                base + lane, 1)
        pltpu.sync_copy(
            slots_hbm.at[pl.ds(sid * A_PER_S + j * 128, 128)], idxbufs[j])

    plsc.subcore_barrier()
    for j in range(A_PER_S // 128):
        pltpu.sync_copy(tokbufs[j], shared_map.at[idxbufs[j]])
    plsc.subcore_barrier()

    base = wid * ROWS_PER_W
    for h in range(2):
        pltpu.sync_copy(
            shared_map.at[pl.ds(base + h * GCHUNK, GCHUNK)], gidx[h])
    g0 = pltpu.async_copy(x_hbm.at[gidx[0]], gbufs[0], gsem[0])
    g1 = pltpu.async_copy(x_hbm.at[gidx[1]], gbufs[1], gsem[1])
    g0.wait()
    w0 = pltpu.async_copy(gbufs[0], out_hbm.at[pl.ds(base, GCHUNK)], wsem[0])
    g1.wait()
    w1 = pltpu.async_copy(
        gbufs[1], out_hbm.at[pl.ds(base + GCHUNK, GCHUNK)], wsem[1])
    w0.wait()
    w1.wait()


@functools.cache
def _sc_dispatch():
    return pl.kernel(
        _sc_gather_body,
        out_type=jax.ShapeDtypeStruct((N_SLOT, D), jnp.float32),
        mesh=plsc.VectorSubcoreMesh(core_axis_name="c", subcore_axis_name="s"),
        compiler_params=pltpu.CompilerParams(needs_layout_passes=False),
        scratch_types=[
            pltpu.VMEM_SHARED((MAP_PAD,), jnp.int32),
            pltpu.VMEM((MAP_PAD // NS,), jnp.int32),
            [pltpu.VMEM((128,), jnp.int32) for _ in range(A_PER_S // 128)],
            [pltpu.VMEM((128,), jnp.int32) for _ in range(A_PER_S // 128)],
            [pltpu.VMEM((GCHUNK,), jnp.int32) for _ in range(2)],
            [pltpu.VMEM((GCHUNK, D), jnp.float32) for _ in range(2)],
            [pltpu.SemaphoreType.DMA for _ in range(2)],
            [pltpu.SemaphoreType.DMA for _ in range(2)],
        ],
    )


def kernel(x, W_gate):
    x_flat = x.reshape(N_TOK, D)
    gates, slots, exp_mask, x_aug = _tc_router(x_flat, W_gate)
    exp_batches = _sc_dispatch()(x_aug, slots.reshape(N_TOK * TOP_K))
    return (gates, exp_mask, exp_batches.reshape(N_EXP, CAP, D))
